# baseline (speedup 1.0000x reference)
import functools

import jax
import jax.numpy as jnp
from jax import lax
from jax.experimental import pallas as pl
from jax.experimental.pallas import tpu as pltpu
from jax.experimental.pallas import tpu_sc as plsc

_N_CLS = 1000
_N_CTX = 16
_DIM = 512
_SEQ = 77
_SUF = _SEQ - 1 - _N_CTX
_ROW = _SEQ * _DIM
_CTX_OFF = _DIM
_SUF_OFF = (1 + _N_CTX) * _DIM


@functools.cache
def _build_sc_kernel():
    info = plsc.get_sparse_core_info()
    nc, ns = info.num_cores, info.num_subcores
    nw = nc * ns
    base_cnt, extra = divmod(_N_CLS, nw)
    mesh = plsc.VectorSubcoreMesh(core_axis_name="c", subcore_axis_name="s")

    @functools.partial(
        pl.kernel,
        out_type=jax.ShapeDtypeStruct((_N_CLS, _ROW), jnp.float32),
        mesh=mesh,
        scratch_types=[
            pltpu.VMEM((_ROW,), jnp.float32),
            pltpu.VMEM((_ROW,), jnp.float32),
        ],
    )
    def prompts_kernel(ctx_hbm, pre_hbm, suf_hbm, out_hbm, buf0, buf1):
        wid = lax.axis_index("s") * nc + lax.axis_index("c")
        cnt = base_cnt + (wid < extra).astype(jnp.int32)
        start = wid * base_cnt + jnp.minimum(wid, extra)
        pltpu.sync_copy(ctx_hbm, buf0.at[pl.ds(_CTX_OFF, _N_CTX * _DIM)])
        pltpu.sync_copy(ctx_hbm, buf1.at[pl.ds(_CTX_OFF, _N_CTX * _DIM)])

        def fill_and_store(c, buf):
            pltpu.sync_copy(pre_hbm.at[c], buf.at[pl.ds(0, _DIM)])
            pltpu.sync_copy(suf_hbm.at[c], buf.at[pl.ds(_SUF_OFF, _SUF * _DIM)])
            pltpu.sync_copy(buf, out_hbm.at[c])

        def body(i, carry):
            c = start + i
            lax.cond(lax.rem(i, 2) == 0,
                     lambda: fill_and_store(c, buf0),
                     lambda: fill_and_store(c, buf1))
            return carry

        lax.fori_loop(0, cnt, body, 0)

    return prompts_kernel


def kernel(ctx, token_prefix, token_suffix):
    out2d = _build_sc_kernel()(
        ctx.reshape(_N_CTX * _DIM),
        token_prefix.reshape(_N_CLS, _DIM),
        token_suffix.reshape(_N_CLS, _SUF * _DIM),
    )
    return out2d.reshape(_N_CLS, _SEQ, _DIM)
